# baseline (device time: 4253383 ns/iter reference)
import jax
import jax.numpy as jnp
from jax import lax
from jax.experimental import pallas as pl
from jax.experimental.pallas import tpu as pltpu


def kernel(x):
    m, n = x.shape

    K = 16
    rows = m // K

    def body(x_ref, out_ref, local_sem, send_sems, recv_sems):
        my_x = lax.axis_index("x")
        my_y = lax.axis_index("y")
        my_z = lax.axis_index("z")
        partner = (1 - my_x, my_y, my_z)

        barrier = pltpu.get_barrier_semaphore()
        pl.semaphore_signal(
            barrier, inc=1, device_id=partner,
            device_id_type=pl.DeviceIdType.MESH,
        )
        pl.semaphore_wait(barrier, 1)

        local = pltpu.make_async_copy(
            x_ref, out_ref.at[pl.ds(my_x * m, m), :], local_sem
        )
        local.start()

        rdmas = []
        for k in range(K):
            rdma = pltpu.make_async_remote_copy(
                src_ref=x_ref.at[pl.ds(k * rows, rows), :],
                dst_ref=out_ref.at[pl.ds(my_x * m + k * rows, rows), :],
                send_sem=send_sems.at[k],
                recv_sem=recv_sems.at[k],
                device_id=partner,
                device_id_type=pl.DeviceIdType.MESH,
            )
            rdma.start()
            rdmas.append(rdma)
        local.wait()
        for rdma in rdmas:
            rdma.wait()

    return pl.pallas_call(
        body,
        out_shape=jax.ShapeDtypeStruct((2 * m, n), jnp.float32),
        in_specs=[pl.BlockSpec(memory_space=pl.ANY)],
        out_specs=pl.BlockSpec(memory_space=pl.ANY),
        scratch_shapes=[
            pltpu.SemaphoreType.DMA,
            pltpu.SemaphoreType.DMA((K,)),
            pltpu.SemaphoreType.DMA((K,)),
        ],
        compiler_params=pltpu.CompilerParams(collective_id=0),
    )(x)


# device time: 1614618 ns/iter; 2.6343x vs baseline; 2.6343x over previous
import jax
import jax.numpy as jnp
from jax import lax
from jax.experimental import pallas as pl
from jax.experimental.pallas import tpu as pltpu


def kernel(x):
    m, n = x.shape
    R = 1024
    K = m // R
    S = 8

    def body(x_ref, out_ref, v, load_sems, store_sems, send_sems, recv_sems):
        my_x = lax.axis_index("x")
        my_y = lax.axis_index("y")
        my_z = lax.axis_index("z")
        partner = (1 - my_x, my_y, my_z)

        barrier = pltpu.get_barrier_semaphore()
        pl.semaphore_signal(
            barrier, inc=1, device_id=partner,
            device_id_type=pl.DeviceIdType.MESH,
        )
        pl.semaphore_wait(barrier, 1)

        stores = {}
        rdmas = {}
        for k in range(K):
            s = k % S
            if k >= S:
                stores[k - S].wait()
                rdmas[k - S].wait_send()
            ld = pltpu.make_async_copy(
                x_ref.at[pl.ds(k * R, R), :], v.at[s], load_sems.at[s]
            )
            ld.start()
            ld.wait()
            st = pltpu.make_async_copy(
                v.at[s],
                out_ref.at[pl.ds(my_x * m + k * R, R), :],
                store_sems.at[s],
            )
            st.start()
            stores[k] = st
            r = pltpu.make_async_remote_copy(
                src_ref=v.at[s],
                dst_ref=out_ref.at[pl.ds(my_x * m + k * R, R), :],
                send_sem=send_sems.at[s],
                recv_sem=recv_sems.at[k],
                device_id=partner,
                device_id_type=pl.DeviceIdType.MESH,
            )
            r.start()
            rdmas[k] = r

        for k in range(K - S, K):
            stores[k].wait()
            rdmas[k].wait_send()
        for k in range(K):
            rdmas[k].wait_recv()

    return pl.pallas_call(
        body,
        out_shape=jax.ShapeDtypeStruct((2 * m, n), jnp.float32),
        in_specs=[pl.BlockSpec(memory_space=pl.ANY)],
        out_specs=pl.BlockSpec(memory_space=pl.ANY),
        scratch_shapes=[
            pltpu.VMEM((S, R, n), jnp.float32),
            pltpu.SemaphoreType.DMA((S,)),
            pltpu.SemaphoreType.DMA((S,)),
            pltpu.SemaphoreType.DMA((S,)),
            pltpu.SemaphoreType.DMA((K,)),
        ],
        compiler_params=pltpu.CompilerParams(collective_id=0),
    )(x)
